# baseline (speedup 1.0000x reference)
import functools

import jax
import jax.numpy as jnp
from jax import lax
from jax.experimental import pallas as pl
from jax.experimental.pallas import tpu as pltpu
from jax.experimental.pallas import tpu_sc as plsc

DIM = 64
BATCH = 4096
HIST = 200
NC = 2
NS = 16
NW = NC * NS
BBLK = BATCH // NW
HC = 2


def _gather_t(idx_flat, table_wide):
    mesh = plsc.VectorSubcoreMesh(core_axis_name="c", subcore_axis_name="s")
    n_chunks = HIST // HC

    @functools.partial(
        pl.kernel,
        out_type=jax.ShapeDtypeStruct((HIST, DIM, BATCH), jnp.float32),
        mesh=mesh,
        scratch_types=[
            pltpu.VMEM((HC * BBLK,), jnp.int32),
            pltpu.VMEM((HC * BBLK, 128), jnp.float32),
            pltpu.VMEM((HC, DIM, BBLK), jnp.float32),
            pltpu.SemaphoreType.DMA,
        ],
        compiler_params=pltpu.CompilerParams(needs_layout_passes=False),
    )
    def gather_kernel(idx_hbm, tab_hbm, out_hbm, idx_v, g_v, o_v, sem):
        wid = lax.axis_index("s") * NC + lax.axis_index("c")
        bbase = wid * BBLK
        iota16 = lax.iota(jnp.int32, 16)

        def chunk_body(ci, carry):
            h0 = ci * HC
            for hl in range(HC):
                pltpu.sync_copy(
                    idx_hbm.at[pl.ds((h0 + hl) * BATCH + bbase, BBLK)],
                    idx_v.at[pl.ds(hl * BBLK, BBLK)])

            pltpu.async_copy(tab_hbm.at[idx_v], g_v, sem).wait()

            for hl in range(HC):
                for grp in range(8):
                    rows = hl * BBLK + grp * 16 + iota16
                    for d in range(DIM):
                        dcol = jnp.full((16,), d, jnp.int32)
                        vals = plsc.load_gather(g_v, [rows, dcol])
                        o_v[hl, d, pl.ds(grp * 16, 16)] = vals

            for hl in range(HC):
                pltpu.sync_copy(
                    o_v.at[hl],
                    out_hbm.at[h0 + hl, :, pl.ds(bbase, BBLK)])
            return carry

        lax.fori_loop(0, n_chunks, chunk_body, 0)

    return gather_kernel(idx_flat, table_wide)


def kernel(x, emb_weight):
    idx_flat = x.T.reshape(BATCH * HIST)
    table_wide = jnp.concatenate([emb_weight, emb_weight], axis=1)
    out3 = _gather_t(idx_flat, table_wide)
    return out3.transpose(2, 0, 1)


# speedup vs baseline: 1.8528x; 1.8528x over previous
import functools

import jax
import jax.numpy as jnp
from jax import lax
from jax.experimental import pallas as pl
from jax.experimental.pallas import tpu as pltpu
from jax.experimental.pallas import tpu_sc as plsc

DIM = 64
NC = 2
NS = 16
NW = NC * NS
CHUNK = 800


@functools.partial(jax.jit, static_argnames=("total",))
def _gather_rows(idx_flat, table, *, total):
    b_per_w = total // NW
    n_chunks = b_per_w // CHUNK
    n_pairs = n_chunks // 2
    mesh = plsc.VectorSubcoreMesh(core_axis_name="c", subcore_axis_name="s")

    @functools.partial(
        pl.kernel,
        out_type=jax.ShapeDtypeStruct((total, DIM), jnp.float32),
        mesh=mesh,
        scratch_types=[
            pltpu.VMEM((b_per_w,), jnp.int32),
            pltpu.VMEM((CHUNK, DIM), jnp.float32),
            pltpu.VMEM((CHUNK, DIM), jnp.float32),
            pltpu.SemaphoreType.DMA,
            pltpu.SemaphoreType.DMA,
        ],
        compiler_params=pltpu.CompilerParams(use_tc_tiling_on_sc=False),
    )
    def gather_kernel(idx_hbm, table_hbm, out_hbm, idx_v, buf0, buf1,
                      sem0, sem1):
        wid = lax.axis_index("s") * NC + lax.axis_index("c")
        base = wid * b_per_w
        pltpu.sync_copy(idx_hbm.at[pl.ds(base, b_per_w)], idx_v)

        def gather(local_off, buf, sem):
            pltpu.async_copy(
                table_hbm.at[idx_v.at[pl.ds(local_off, CHUNK)]], buf, sem)

        def wait(buf, sem):
            pltpu.make_async_copy(table_hbm.at[pl.ds(0, CHUNK)], buf,
                                  sem).wait()

        gather(0, buf0, sem0)

        def body(j, carry):
            c0 = 2 * j * CHUNK
            gather(c0 + CHUNK, buf1, sem1)
            wait(buf0, sem0)
            pltpu.sync_copy(buf0, out_hbm.at[pl.ds(base + c0, CHUNK)])

            @pl.when(j + 1 < n_pairs)
            def _():
                gather(c0 + 2 * CHUNK, buf0, sem0)

            wait(buf1, sem1)
            pltpu.sync_copy(buf1,
                            out_hbm.at[pl.ds(base + c0 + CHUNK, CHUNK)])
            return carry

        lax.fori_loop(0, n_pairs, body, 0)

    return gather_kernel(idx_flat, table)


def kernel(x, emb_weight):
    b, h = x.shape
    total = b * h
    out = _gather_rows(x.reshape(total), emb_weight, total=total)
    return out.reshape(b, h, DIM)


# speedup vs baseline: 1.8531x; 1.0002x over previous
import functools

import jax
import jax.numpy as jnp
from jax import lax
from jax.experimental import pallas as pl
from jax.experimental.pallas import tpu as pltpu
from jax.experimental.pallas import tpu_sc as plsc

DIM = 64
NC = 2
NS = 16
NW = NC * NS
CHUNK = 512


@functools.partial(jax.jit, static_argnames=("total",))
def _gather_rows(idx_flat, table, *, total):
    b_per_w = total // NW
    n_chunks = b_per_w // CHUNK
    n_trips = (n_chunks - 2) // 3
    mesh = plsc.VectorSubcoreMesh(core_axis_name="c", subcore_axis_name="s")

    @functools.partial(
        pl.kernel,
        out_type=jax.ShapeDtypeStruct((total, DIM), jnp.float32),
        mesh=mesh,
        scratch_types=[
            pltpu.VMEM((b_per_w,), jnp.int32),
            pltpu.VMEM((CHUNK, DIM), jnp.float32),
            pltpu.VMEM((CHUNK, DIM), jnp.float32),
            pltpu.VMEM((CHUNK, DIM), jnp.float32),
            pltpu.SemaphoreType.DMA,
            pltpu.SemaphoreType.DMA,
            pltpu.SemaphoreType.DMA,
        ],
        compiler_params=pltpu.CompilerParams(use_tc_tiling_on_sc=False),
    )
    def gather_kernel(idx_hbm, table_hbm, out_hbm, idx_v, b0, b1, b2,
                      sem0, sem1, sem2):
        wid = lax.axis_index("s") * NC + lax.axis_index("c")
        base = wid * b_per_w
        pltpu.sync_copy(idx_hbm.at[pl.ds(base, b_per_w)], idx_v)

        bufs = (b0, b1, b2)
        sems = (sem0, sem1, sem2)

        def gather(c, buf, sem):
            pltpu.async_copy(
                table_hbm.at[idx_v.at[pl.ds(c * CHUNK, CHUNK)]], buf, sem)

        def wait(buf, sem):
            pltpu.make_async_copy(table_hbm.at[pl.ds(0, CHUNK)], buf,
                                  sem).wait()

        def store(c, buf):
            pltpu.sync_copy(buf,
                            out_hbm.at[pl.ds(base + c * CHUNK, CHUNK)])

        gather(0, b0, sem0)
        gather(1, b1, sem1)

        def body(j, carry):
            c = 3 * j
            for k in range(3):
                bk, sk = bufs[k], sems[k]
                nb, nsem = bufs[(k + 2) % 3], sems[(k + 2) % 3]
                wait(bk, sk)
                gather(c + k + 2, nb, nsem)
                store(c + k, bk)
            return carry

        lax.fori_loop(0, n_trips, body, 0)

        c_tail = 3 * n_trips
        wait(b0, sem0)
        store(c_tail, b0)
        wait(b1, sem1)
        store(c_tail + 1, b1)

    return gather_kernel(idx_flat, table)


def kernel(x, emb_weight):
    b, h = x.shape
    total = b * h
    out = _gather_rows(x.reshape(total), emb_weight, total=total)
    return out.reshape(b, h, DIM)
